# Initial kernel scaffold; baseline (speedup 1.0000x reference)
import jax
import jax.numpy as jnp
from jax.experimental import pallas as pl


def kernel(x, W_qkv, b_qkv, W_lepe, b_lepe, W_o, b_o):
    raise NotImplementedError("write your pallas kernel here")


# speedup vs baseline: 1.0263x; 1.0263x over previous
import jax
import jax.numpy as jnp
from jax import lax
from jax.experimental import pallas as pl
from jax.experimental.pallas import tpu as pltpu

C_DIM = 192
QK = 192
HEADS = 8
CH = QK // HEADS
NWIN = 7
TOPK_N = 8
WH = 8
W2 = WH * WH
P2 = NWIN * NWIN
SIDE_N = 7
PAD = SIDE_N // 2
HW = NWIN * WH
KVC = QK + C_DIM
SCALE = QK ** -0.5


def kernel(x, W_qkv, b_qkv, W_lepe, b_lepe, W_o, b_o):
    B, n, _ = x.shape
    x_sp = x.reshape(B, HW, HW, C_DIM)
    wl = jnp.transpose(W_lepe[:, 0], (1, 2, 0))
    b_qkv2 = b_qkv.reshape(1, -1)
    bl2 = b_lepe.reshape(1, C_DIM)
    bo2 = b_o.reshape(1, C_DIM)

    def _proj_body(x_ref, w_ref, b_ref, q_ref, kv_ref, v_ref, qm_ref, km_ref):
        x2 = x_ref[0].reshape(W2, C_DIM)
        qkv = jnp.dot(x2, w_ref[...], preferred_element_type=jnp.float32) + b_ref[0]
        q = qkv[:, :QK]
        k = qkv[:, QK:2 * QK]
        v = qkv[:, 2 * QK:]
        q_ref[0, 0] = q
        kv_ref[0, 0] = qkv[:, QK:]
        v_ref[0] = v.reshape(WH, WH, C_DIM)
        qm_ref[0, 0, 0] = jnp.sum(q, axis=0) * (1.0 / W2)
        km_ref[0, 0, 0] = jnp.sum(k, axis=0) * (1.0 / W2)

    q_wm, kv_wm, v_sp, qm, km = pl.pallas_call(
        _proj_body,
        grid=(B, NWIN, NWIN),
        in_specs=[
            pl.BlockSpec((1, WH, WH, C_DIM), lambda b, j, i: (b, j, i, 0)),
            pl.BlockSpec((C_DIM, 3 * QK), lambda b, j, i: (0, 0)),
            pl.BlockSpec((1, 3 * QK), lambda b, j, i: (0, 0)),
        ],
        out_specs=[
            pl.BlockSpec((1, 1, W2, QK), lambda b, j, i: (b, j * NWIN + i, 0, 0)),
            pl.BlockSpec((1, 1, W2, KVC), lambda b, j, i: (b, j * NWIN + i, 0, 0)),
            pl.BlockSpec((1, WH, WH, C_DIM), lambda b, j, i: (b, j, i, 0)),
            pl.BlockSpec((1, 1, 1, QK), lambda b, j, i: (b, j * NWIN + i, 0, 0)),
            pl.BlockSpec((1, 1, 1, QK), lambda b, j, i: (b, j * NWIN + i, 0, 0)),
        ],
        out_shape=[
            jax.ShapeDtypeStruct((B, P2, W2, QK), jnp.float32),
            jax.ShapeDtypeStruct((B, P2, W2, KVC), jnp.float32),
            jax.ShapeDtypeStruct((B, HW, HW, C_DIM), jnp.float32),
            jax.ShapeDtypeStruct((B, P2, 1, QK), jnp.float32),
            jax.ShapeDtypeStruct((B, P2, 1, QK), jnp.float32),
        ],
    )(x_sp, W_qkv, b_qkv2)

    def _route_body(qm_ref, km_ref, idx_ref):
        for b in range(B):
            qmb = qm_ref[b].reshape(P2, QK) * SCALE
            kmb = km_ref[b].reshape(P2, QK)
            logit = lax.dot_general(qmb, kmb, (((1,), (1,)), ((), ())),
                                    preferred_element_type=jnp.float32)
            iota = lax.broadcasted_iota(jnp.int32, (P2, P2), 1)
            l = logit
            cols = []
            for _ in range(TOPK_N):
                m = jnp.max(l, axis=1, keepdims=True)
                cand = jnp.where(l >= m, iota, 2 * P2)
                sel = jnp.min(cand, axis=1, keepdims=True)
                cols.append(sel)
                l = jnp.where(iota == sel, -1e30, l)
            idx_ref[b] = jnp.concatenate(cols, axis=1)

    r_idx = pl.pallas_call(
        _route_body,
        out_shape=jax.ShapeDtypeStruct((B, P2, TOPK_N), jnp.int32),
    )(qm, km)

    def _lepe_body(v_ref, wl_ref, bl_ref, out_ref, pad_ref):
        pad_ref[...] = jnp.zeros((HW + 2 * PAD, HW + 2 * PAD, C_DIM), jnp.float32)
        pad_ref[PAD:PAD + HW, PAD:PAD + HW, :] = v_ref[0]
        for ys in range(0, HW, WH):
            acc = jnp.zeros((WH, HW, C_DIM), jnp.float32)
            for dy in range(SIDE_N):
                row = pad_ref[ys + dy:ys + dy + WH, :, :]
                for dx in range(SIDE_N):
                    acc += row[:, dx:dx + HW, :] * wl_ref[dy, dx]
            out_ref[0, ys:ys + WH] = acc + bl_ref[0]

    lepe_sp = pl.pallas_call(
        _lepe_body,
        grid=(B,),
        in_specs=[
            pl.BlockSpec((1, HW, HW, C_DIM), lambda b: (b, 0, 0, 0)),
            pl.BlockSpec((SIDE_N, SIDE_N, C_DIM), lambda b: (0, 0, 0)),
            pl.BlockSpec((1, C_DIM), lambda b: (0, 0)),
        ],
        out_specs=pl.BlockSpec((1, HW, HW, C_DIM), lambda b: (b, 0, 0, 0)),
        out_shape=jax.ShapeDtypeStruct((B, HW, HW, C_DIM), jnp.float32),
        scratch_shapes=[pltpu.VMEM((HW + 2 * PAD, HW + 2 * PAD, C_DIM), jnp.float32)],
    )(v_sp, wl, bl2)

    def _attn_body(idx_ref, q_ref, kv_ref, lepe_ref, wo_ref, bo_ref, out_ref):
        b = pl.program_id(0)
        w = pl.program_id(1)
        q = q_ref[0, 0]
        ks = []
        vs = []
        for t in range(TOPK_N):
            r = idx_ref[b, w, t]
            kvw = kv_ref[0, pl.ds(r, 1)]
            ks.append(kvw[0, :, :QK])
            vs.append(kvw[0, :, QK:])
        kk = jnp.concatenate(ks, axis=0)
        vv = jnp.concatenate(vs, axis=0)
        outs = []
        for hh in range(HEADS):
            sl = slice(hh * CH, (hh + 1) * CH)
            s = lax.dot_general(q[:, sl], kk[:, sl], (((1,), (1,)), ((), ())),
                                preferred_element_type=jnp.float32) * SCALE
            m = jnp.max(s, axis=1, keepdims=True)
            e = jnp.exp(s - m)
            p = e / jnp.sum(e, axis=1, keepdims=True)
            outs.append(jnp.dot(p, vv[:, sl], preferred_element_type=jnp.float32))
        o = jnp.concatenate(outs, axis=1) + lepe_ref[0].reshape(W2, C_DIM)
        res = jnp.dot(o, wo_ref[...], preferred_element_type=jnp.float32) + bo_ref[0]
        out_ref[0] = res.reshape(WH, WH, C_DIM)

    grid_spec = pltpu.PrefetchScalarGridSpec(
        num_scalar_prefetch=1,
        grid=(B, P2),
        in_specs=[
            pl.BlockSpec((1, 1, W2, QK), lambda b, w, idx: (b, w, 0, 0)),
            pl.BlockSpec((1, P2, W2, KVC), lambda b, w, idx: (b, 0, 0, 0)),
            pl.BlockSpec((1, WH, WH, C_DIM), lambda b, w, idx: (b, w // NWIN, w % NWIN, 0)),
            pl.BlockSpec((C_DIM, C_DIM), lambda b, w, idx: (0, 0)),
            pl.BlockSpec((1, C_DIM), lambda b, w, idx: (0, 0)),
        ],
        out_specs=pl.BlockSpec((1, WH, WH, C_DIM), lambda b, w, idx: (b, w // NWIN, w % NWIN, 0)),
    )
    out_sp = pl.pallas_call(
        _attn_body,
        grid_spec=grid_spec,
        out_shape=jax.ShapeDtypeStruct((B, HW, HW, C_DIM), jnp.float32),
    )(r_idx, q_wm, kv_wm, lepe_sp, W_o, bo2)

    return out_sp.reshape(B, n, C_DIM)
